# baseline (speedup 1.0000x reference)
import functools

import jax
import jax.numpy as jnp
from jax.experimental import pallas as pl
from jax.experimental.pallas import tpu as pltpu


def _scores_body(qe_ref, wt_ref, b_ref, tab_ref, out_ref):
    proj = jnp.dot(qe_ref[...], wt_ref[...], preferred_element_type=jnp.float32)
    proj = proj + b_ref[...]
    qn = proj / jnp.maximum(
        jnp.sqrt(jnp.sum(proj * proj, axis=-1, keepdims=True)), 1e-12)
    tab = tab_ref[...]
    tabn = tab / jnp.maximum(
        jnp.sqrt(jnp.sum(tab * tab, axis=-1, keepdims=True)), 1e-12)
    out_ref[...] = jax.lax.dot_general(
        qn, tabn, (((1,), (1,)), ((), ())),
        preferred_element_type=jnp.float32)


@functools.partial(jax.jit, static_argnames=("bq", "bk"))
def _scores(qe, tab_pad, wt, b, bq=256, bk=2048):
    Q, dq = qe.shape
    K_pad, dt = tab_pad.shape
    grid = (Q // bq, K_pad // bk)
    return pl.pallas_call(
        _scores_body,
        grid=grid,
        in_specs=[
            pl.BlockSpec((bq, dq), lambda i, j: (i, 0)),
            pl.BlockSpec((dq, dt), lambda i, j: (0, 0)),
            pl.BlockSpec((1, dt), lambda i, j: (0, 0)),
            pl.BlockSpec((bk, dt), lambda i, j: (j, 0)),
        ],
        out_specs=pl.BlockSpec((bq, bk), lambda i, j: (i, j)),
        out_shape=jax.ShapeDtypeStruct((Q, K_pad), jnp.float32),
        compiler_params=pltpu.CompilerParams(
            dimension_semantics=("arbitrary", "arbitrary"),
        ),
    )(qe, wt, b.reshape(1, dt), tab_pad)


def kernel(query_embedding, movie_tag_embeddings, W, b, k):
    K, dt = movie_tag_embeddings.shape
    bk = 2048
    K_pad = ((K + bk - 1) // bk) * bk
    tab_pad = jnp.concatenate(
        [movie_tag_embeddings,
         jnp.zeros((K_pad - K, dt), movie_tag_embeddings.dtype)], axis=0)
    scores = _scores(query_embedding, tab_pad, W.T, b)
    vals, idx = jax.lax.top_k(scores[:, :K], 100)
    return vals, idx


# speedup vs baseline: 8.8167x; 8.8167x over previous
import functools

import jax
import jax.numpy as jnp
from jax import lax
from jax.experimental import pallas as pl
from jax.experimental.pallas import tpu as pltpu
from jax.experimental.pallas import tpu_sc as plsc

Q = 4096
DQ = 384
DT = 128
K_REAL = 100000
GSZ = 32
NG = 3200
K_PAD = NG * GSZ
BQ = 256
BK = 4096
NKC = K_PAD // BK
TOPK = 100
CBUF = 512
GL = 128
SG = 128
NG2 = K_PAD // SG
NG2P = 896
NW = 32
QPW = Q // NW


def _l2n(x):
    n = jnp.sqrt(jnp.sum(x * x, axis=-1, keepdims=True))
    return x / jnp.maximum(n, 1e-12)


def _k1_body(qn_ref, tab_ref, s_ref, m_ref):
    j = pl.program_id(1)
    s = lax.dot_general(qn_ref[...], tab_ref[...], (((1,), (1,)), ((), ())),
                        preferred_element_type=jnp.float32)
    col = j * BK + lax.broadcasted_iota(jnp.int32, (BQ, BK), 1)
    s = jnp.where(col < K_REAL, s, -2.0)
    s_ref[...] = s
    m_ref[...] = jnp.max(s.reshape(BQ, BK // GSZ, GSZ), axis=2)


@jax.jit
def _k1(qn, tabn_pad):
    return pl.pallas_call(
        _k1_body,
        grid=(Q // BQ, NKC),
        in_specs=[
            pl.BlockSpec((BQ, DT), lambda i, j: (i, 0)),
            pl.BlockSpec((BK, DT), lambda i, j: (j, 0)),
        ],
        out_specs=[
            pl.BlockSpec((BQ, BK), lambda i, j: (i, j)),
            pl.BlockSpec((BQ, BK // GSZ), lambda i, j: (i, j)),
        ],
        out_shape=[
            jax.ShapeDtypeStruct((Q, K_PAD), jnp.float32),
            jax.ShapeDtypeStruct((Q, NG), jnp.float32),
        ],
        compiler_params=pltpu.CompilerParams(
            dimension_semantics=("arbitrary", "arbitrary"),
        ),
    )(qn, tabn_pad)


BQ2 = 64


def _k2_body(m_ref, m128_ref):
    mb = lax.bitcast_convert_type(m_ref[...], jnp.int32)
    mkey = jnp.where(mb >= 0, mb,
                     jnp.bitwise_xor(jnp.bitwise_not(mb), jnp.int32(-2**31)))

    def step(it, u):
        bit = 31 - it
        ucand = jnp.bitwise_or(u, jnp.int32(1) << bit)
        tcand = jnp.bitwise_xor(ucand, jnp.int32(-2**31))
        cnt = jnp.sum((mkey >= tcand).astype(jnp.int32), axis=1, keepdims=True)
        return jnp.where(cnt >= TOPK, ucand, u)

    u = lax.fori_loop(0, 32, step, jnp.zeros((BQ2, 1), jnp.int32))
    res = jnp.bitwise_xor(u, jnp.int32(-2**31))
    fb = jnp.where(res >= 0, res,
                   jnp.bitwise_not(jnp.bitwise_xor(res, jnp.int32(-2**31))))
    tau = lax.bitcast_convert_type(fb, jnp.float32)
    m128 = jnp.max(m_ref[...].reshape(BQ2, NG2, SG // GSZ), axis=2)
    m128_ref[...] = jnp.concatenate(
        [m128, jnp.full((BQ2, NG2P - NG2 - 1), -2.0, jnp.float32),
         tau], axis=1)


@jax.jit
def _k2(m):
    return pl.pallas_call(
        _k2_body,
        grid=(Q // BQ2,),
        in_specs=[pl.BlockSpec((BQ2, NG), lambda i: (i, 0))],
        out_specs=pl.BlockSpec((BQ2, NG2P), lambda i: (i, 0)),
        out_shape=jax.ShapeDtypeStruct((Q, NG2P), jnp.float32),
    )(m)


def _k3_make():
    mesh = plsc.VectorSubcoreMesh(core_axis_name="c", subcore_axis_name="s")

    @functools.partial(
        pl.kernel,
        mesh=mesh,
        compiler_params=pltpu.CompilerParams(needs_layout_passes=False),
        out_type=[
            jax.ShapeDtypeStruct((Q, CBUF), jnp.float32),
            jax.ShapeDtypeStruct((Q, CBUF), jnp.int32),
        ],
        scratch_types=[
            pltpu.VMEM((NG2P,), jnp.float32),
            pltpu.VMEM((GL + 16,), jnp.int32),
            pltpu.VMEM((GL + 16, SG), jnp.float32),
            pltpu.VMEM((CBUF + 32,), jnp.float32),
            pltpu.VMEM((CBUF + 32,), jnp.int32),
            pltpu.SemaphoreType.DMA,
        ],
    )
    def k3(m_hbm, s2_hbm, cval_hbm, cidx_hbm,
           m_v, glist_v, grow_v, cval_v, cidx_v, sem):
        wid = lax.axis_index("s") * 2 + lax.axis_index("c")
        q0 = wid * QPW
        iota = lax.iota(jnp.int32, 16)

        def per_query(i, _):
            q = q0 + i
            pltpu.sync_copy(m_hbm.at[q], m_v)
            tau_b = jnp.broadcast_to(m_v[pl.ds(NG2P - 16, 16)][15], (16,))
            pad_row = q * NG2 + NG2 - 8

            def prefill_g(j, _c):
                glist_v[pl.ds(j * 16, 16)] = jnp.full((16,), pad_row, jnp.int32)
                return _c
            lax.fori_loop(0, (GL + 16) // 16, prefill_g, 0)

            def scan_m(j, cnt):
                v = m_v[pl.ds(j * 16, 16)]
                msk = v >= tau_b
                gid = q * NG2 + j * 16 + iota
                off = jnp.minimum(cnt, GL - 16)
                incl = plsc.cumsum(msk.astype(jnp.int32))
                pos = jnp.where(msk, off + incl - 1, GL + 8)
                plsc.store_scatter(glist_v, [pos], gid)
                npop = jnp.max(plsc.all_reduce_population_count(msk))
                return cnt + npop
            lax.fori_loop(0, NG2 // 16, scan_m, jnp.int32(0))

            pltpu.async_copy(s2_hbm.at[glist_v], grow_v, sem).wait()

            def prefill_c(j, _c):
                cval_v[pl.ds(j * 16, 16)] = jnp.full((16,), -2.0, jnp.float32)
                cidx_v[pl.ds(j * 16, 16)] = jnp.zeros((16,), jnp.int32)
                return _c
            lax.fori_loop(0, (CBUF + 32) // 16, prefill_c, 0)

            def scan_rows(jj, ccnt):
                gvec = glist_v[pl.ds(jj * 16, 16)]
                cc = ccnt
                for l in range(16):
                    base = (gvec[l] - q * NG2) * SG
                    row = jj * 16 + l
                    for h in range(SG // 16):
                        v = grow_v[row, pl.ds(h * 16, 16)]
                        msk = v >= tau_b
                        idxv = base + h * 16 + iota
                        off = jnp.minimum(cc, CBUF)
                        incl = plsc.cumsum(msk.astype(jnp.int32))
                        pos = jnp.where(msk, off + incl - 1, CBUF + 24)
                        plsc.store_scatter(cval_v, [pos], v)
                        plsc.store_scatter(cidx_v, [pos], idxv)
                        cc = cc + jnp.max(plsc.all_reduce_population_count(msk))
                return cc
            lax.fori_loop(0, GL // 16, scan_rows, jnp.int32(0))

            pltpu.sync_copy(cval_v.at[pl.ds(0, CBUF)], cval_hbm.at[q])
            pltpu.sync_copy(cidx_v.at[pl.ds(0, CBUF)], cidx_hbm.at[q])
            return _

        lax.fori_loop(0, QPW, per_query, 0)

    return k3


_K3 = None


def _get_k3():
    global _K3
    if _K3 is None:
        _K3 = _k3_make()
    return _K3


def _k4_body(cv_ref, ci_ref, ov_ref, oi_ref):
    v0 = cv_ref[...]
    ci = ci_ref[...]
    il = lax.broadcasted_iota(jnp.int32, (BQ, CBUF), 1)
    ol = lax.broadcasted_iota(jnp.int32, (BQ, 128), 1)

    def step(t, carry):
        v, ov, oi = carry
        m = jnp.max(v, axis=1, keepdims=True)
        p = jnp.min(jnp.where(v == m, il, CBUF), axis=1, keepdims=True)
        hit = il == p
        idx = jnp.sum(jnp.where(hit, ci, 0), axis=1, keepdims=True)
        v = jnp.where(hit, -3.0, v)
        ov = jnp.where(ol == t, m, ov)
        oi = jnp.where(ol == t, idx, oi)
        return v, ov, oi

    _, ov, oi = lax.fori_loop(
        0, TOPK, step,
        (v0, jnp.full((BQ, 128), -3.0, jnp.float32),
         jnp.zeros((BQ, 128), jnp.int32)))
    ov_ref[...] = ov
    oi_ref[...] = oi


@jax.jit
def _k4(cv, ci):
    return pl.pallas_call(
        _k4_body,
        grid=(Q // BQ,),
        in_specs=[
            pl.BlockSpec((BQ, CBUF), lambda i: (i, 0)),
            pl.BlockSpec((BQ, CBUF), lambda i: (i, 0)),
        ],
        out_specs=[
            pl.BlockSpec((BQ, 128), lambda i: (i, 0)),
            pl.BlockSpec((BQ, 128), lambda i: (i, 0)),
        ],
        out_shape=[
            jax.ShapeDtypeStruct((Q, 128), jnp.float32),
            jax.ShapeDtypeStruct((Q, 128), jnp.int32),
        ],
    )(cv, ci)


def kernel(query_embedding, movie_tag_embeddings, W, b, k):
    tabn = _l2n(movie_tag_embeddings)
    tabn_pad = jnp.concatenate(
        [tabn, jnp.zeros((K_PAD - K_REAL, DT), tabn.dtype)], axis=0)
    qn = _l2n(query_embedding @ W.T + b)
    scores, m = _k1(qn, tabn_pad)
    m128 = _k2(m)
    s2 = scores.reshape(Q * NG2, SG)
    cv, ci = _get_k3()(m128, s2)
    ov, oi = _k4(cv, ci)
    return ov[:, :TOPK], oi[:, :TOPK]


# speedup vs baseline: 9.2380x; 1.0478x over previous
import functools

import jax
import jax.numpy as jnp
from jax import lax
from jax.experimental import pallas as pl
from jax.experimental.pallas import tpu as pltpu
from jax.experimental.pallas import tpu_sc as plsc

Q = 4096
DQ = 384
DT = 128
K_REAL = 100000
GSZ = 32
NG = 3200
K_PAD = NG * GSZ
BQ = 256
BK = 4096
NKC = K_PAD // BK
TOPK = 100
CBUF = 512
GL = 128
SG = 128
NG2 = K_PAD // SG
NG2P = 896
NW = 32
QPW = Q // NW


def _l2n(x):
    n = jnp.sqrt(jnp.sum(x * x, axis=-1, keepdims=True))
    return x / jnp.maximum(n, 1e-12)


def _k1_body(qn_ref, tab_ref, s_ref, m_ref):
    j = pl.program_id(1)
    s = lax.dot_general(qn_ref[...], tab_ref[...], (((1,), (1,)), ((), ())),
                        preferred_element_type=jnp.float32)
    col = j * BK + lax.broadcasted_iota(jnp.int32, (BQ, BK), 1)
    s = jnp.where(col < K_REAL, s, -2.0)
    s_ref[...] = s
    m_ref[...] = jnp.max(s.reshape(BQ, BK // GSZ, GSZ), axis=2)


@jax.jit
def _k1(qn, tabn_pad):
    return pl.pallas_call(
        _k1_body,
        grid=(Q // BQ, NKC),
        in_specs=[
            pl.BlockSpec((BQ, DT), lambda i, j: (i, 0)),
            pl.BlockSpec((BK, DT), lambda i, j: (j, 0)),
        ],
        out_specs=[
            pl.BlockSpec((BQ, BK), lambda i, j: (i, j)),
            pl.BlockSpec((BQ, BK // GSZ), lambda i, j: (i, j)),
        ],
        out_shape=[
            jax.ShapeDtypeStruct((Q, K_PAD), jnp.float32),
            jax.ShapeDtypeStruct((Q, NG), jnp.float32),
        ],
        compiler_params=pltpu.CompilerParams(
            dimension_semantics=("arbitrary", "arbitrary"),
        ),
    )(qn, tabn_pad)


BQ2 = 64


def _k2_body(m_ref, m128_ref):
    mb = lax.bitcast_convert_type(m_ref[...], jnp.int32)
    mkey = jnp.where(mb >= 0, mb,
                     jnp.bitwise_xor(jnp.bitwise_not(mb), jnp.int32(-2**31)))

    def step(it, u):
        bit = 31 - it
        ucand = jnp.bitwise_or(u, jnp.int32(1) << bit)
        tcand = jnp.bitwise_xor(ucand, jnp.int32(-2**31))
        cnt = jnp.sum((mkey >= tcand).astype(jnp.int32), axis=1, keepdims=True)
        return jnp.where(cnt >= TOPK, ucand, u)

    u = lax.fori_loop(0, 32, step, jnp.zeros((BQ2, 1), jnp.int32))
    res = jnp.bitwise_xor(u, jnp.int32(-2**31))
    fb = jnp.where(res >= 0, res,
                   jnp.bitwise_not(jnp.bitwise_xor(res, jnp.int32(-2**31))))
    tau = lax.bitcast_convert_type(fb, jnp.float32)
    m128 = jnp.max(m_ref[...].reshape(BQ2, NG2, SG // GSZ), axis=2)
    m128_ref[...] = jnp.concatenate(
        [m128, jnp.full((BQ2, NG2P - NG2 - 1), -2.0, jnp.float32),
         tau], axis=1)


@jax.jit
def _k2(m):
    return pl.pallas_call(
        _k2_body,
        grid=(Q // BQ2,),
        in_specs=[pl.BlockSpec((BQ2, NG), lambda i: (i, 0))],
        out_specs=pl.BlockSpec((BQ2, NG2P), lambda i: (i, 0)),
        out_shape=jax.ShapeDtypeStruct((Q, NG2P), jnp.float32),
    )(m)


def _k3_make():
    mesh = plsc.VectorSubcoreMesh(core_axis_name="c", subcore_axis_name="s")

    @functools.partial(
        pl.kernel,
        mesh=mesh,
        compiler_params=pltpu.CompilerParams(needs_layout_passes=False),
        out_type=[
            jax.ShapeDtypeStruct((Q, CBUF), jnp.float32),
            jax.ShapeDtypeStruct((Q, CBUF), jnp.int32),
        ],
        scratch_types=[
            pltpu.VMEM((NG2P,), jnp.float32),
            pltpu.VMEM((GL + 16,), jnp.int32),
            pltpu.VMEM((GL + 16, SG), jnp.float32),
            pltpu.VMEM((CBUF + 32,), jnp.float32),
            pltpu.VMEM((CBUF + 32,), jnp.int32),
            pltpu.SemaphoreType.DMA,
        ],
    )
    def k3(m_hbm, s2_hbm, cval_hbm, cidx_hbm,
           m_v, glist_v, grow_v, cval_v, cidx_v, sem):
        wid = lax.axis_index("s") * 2 + lax.axis_index("c")
        q0 = wid * QPW
        iota = lax.iota(jnp.int32, 16)

        def per_query(i, _):
            q = q0 + i
            pltpu.sync_copy(m_hbm.at[q], m_v)
            tau_b = jnp.broadcast_to(m_v[pl.ds(NG2P - 16, 16)][15], (16,))
            pad_row = q * NG2 + NG2 - 8

            def prefill_g(j, _c):
                glist_v[pl.ds(j * 16, 16)] = jnp.full((16,), pad_row, jnp.int32)
                return _c
            lax.fori_loop(0, (GL + 16) // 16, prefill_g, 0)

            def scan_m(j, cnt):
                v = m_v[pl.ds(j * 16, 16)]
                msk = v >= tau_b
                gid = q * NG2 + j * 16 + iota
                off = jnp.minimum(cnt, GL - 16)
                incl = plsc.cumsum(msk.astype(jnp.int32))
                pos = jnp.where(msk, off + incl - 1, GL + 8)
                plsc.store_scatter(glist_v, [pos], gid)
                npop = plsc.all_reduce_population_count(msk)[0]
                return cnt + npop
            lax.fori_loop(0, NG2 // 16, scan_m, jnp.int32(0))

            pltpu.async_copy(s2_hbm.at[glist_v], grow_v, sem).wait()

            def prefill_c(j, _c):
                cval_v[pl.ds(j * 16, 16)] = jnp.full((16,), -2.0, jnp.float32)
                cidx_v[pl.ds(j * 16, 16)] = jnp.zeros((16,), jnp.int32)
                return _c
            lax.fori_loop(0, (CBUF + 32) // 16, prefill_c, 0)

            def scan_rows(jj, ccnt):
                gvec = glist_v[pl.ds(jj * 16, 16)]
                cc = ccnt
                for l in range(16):
                    base = (gvec[l] - q * NG2) * SG
                    row = jj * 16 + l
                    for h in range(SG // 16):
                        v = grow_v[row, pl.ds(h * 16, 16)]
                        msk = v >= tau_b
                        idxv = base + h * 16 + iota
                        off = jnp.minimum(cc, CBUF)
                        incl = plsc.cumsum(msk.astype(jnp.int32))
                        pos = jnp.where(msk, off + incl - 1, CBUF + 24)
                        plsc.store_scatter(cval_v, [pos], v)
                        plsc.store_scatter(cidx_v, [pos], idxv)
                        cc = cc + plsc.all_reduce_population_count(msk)[0]
                return cc
            lax.fori_loop(0, GL // 16, scan_rows, jnp.int32(0))

            pltpu.sync_copy(cval_v.at[pl.ds(0, CBUF)], cval_hbm.at[q])
            pltpu.sync_copy(cidx_v.at[pl.ds(0, CBUF)], cidx_hbm.at[q])
            return _

        lax.fori_loop(0, QPW, per_query, 0)

    return k3


_K3 = None


def _get_k3():
    global _K3
    if _K3 is None:
        _K3 = _k3_make()
    return _K3


def _k4_body(cv_ref, ci_ref, ov_ref, oi_ref):
    v0 = cv_ref[...]
    ci = ci_ref[...]
    il = lax.broadcasted_iota(jnp.int32, (BQ, CBUF), 1)
    ol = lax.broadcasted_iota(jnp.int32, (BQ, 128), 1)

    def step(t, carry):
        v, ov, oi = carry
        m = jnp.max(v, axis=1, keepdims=True)
        p = jnp.min(jnp.where(v == m, il, CBUF), axis=1, keepdims=True)
        hit = il == p
        idx = jnp.sum(jnp.where(hit, ci, 0), axis=1, keepdims=True)
        v = jnp.where(hit, -3.0, v)
        ov = jnp.where(ol == t, m, ov)
        oi = jnp.where(ol == t, idx, oi)
        return v, ov, oi

    _, ov, oi = lax.fori_loop(
        0, TOPK, step,
        (v0, jnp.full((BQ, 128), -3.0, jnp.float32),
         jnp.zeros((BQ, 128), jnp.int32)))
    ov_ref[...] = ov
    oi_ref[...] = oi


@jax.jit
def _k4(cv, ci):
    return pl.pallas_call(
        _k4_body,
        grid=(Q // BQ,),
        in_specs=[
            pl.BlockSpec((BQ, CBUF), lambda i: (i, 0)),
            pl.BlockSpec((BQ, CBUF), lambda i: (i, 0)),
        ],
        out_specs=[
            pl.BlockSpec((BQ, 128), lambda i: (i, 0)),
            pl.BlockSpec((BQ, 128), lambda i: (i, 0)),
        ],
        out_shape=[
            jax.ShapeDtypeStruct((Q, 128), jnp.float32),
            jax.ShapeDtypeStruct((Q, 128), jnp.int32),
        ],
    )(cv, ci)


def kernel(query_embedding, movie_tag_embeddings, W, b, k):
    tabn = _l2n(movie_tag_embeddings)
    tabn_pad = jnp.concatenate(
        [tabn, jnp.zeros((K_PAD - K_REAL, DT), tabn.dtype)], axis=0)
    qn = _l2n(query_embedding @ W.T + b)
    scores, m = _k1(qn, tabn_pad)
    m128 = _k2(m)
    s2 = scores.reshape(Q * NG2, SG)
    cv, ci = _get_k3()(m128, s2)
    ov, oi = _k4(cv, ci)
    return ov[:, :TOPK], oi[:, :TOPK]


# speedup vs baseline: 14.3530x; 1.5537x over previous
import functools

import jax
import jax.numpy as jnp
from jax import lax
from jax.experimental import pallas as pl
from jax.experimental.pallas import tpu as pltpu
from jax.experimental.pallas import tpu_sc as plsc

Q = 4096
DQ = 384
DT = 128
K_REAL = 100000
GSZ = 32
NG = 3200
K_PAD = NG * GSZ
BQ = 256
BK = 4096
NKC = K_PAD // BK
TOPK = 100
CBUF = 512
GL = 128
SG = 128
NG2 = K_PAD // SG
NG2P = 896
MROW = 3328
NW = 32
QPW = Q // NW


def _l2n(x):
    n = jnp.sqrt(jnp.sum(x * x, axis=-1, keepdims=True))
    return x / jnp.maximum(n, 1e-12)


def _k1_body(qn_ref, tab_ref, s_ref, m_ref):
    j = pl.program_id(1)
    s = lax.dot_general(qn_ref[...], tab_ref[...], (((1,), (1,)), ((), ())),
                        preferred_element_type=jnp.float32)
    col = j * BK + lax.broadcasted_iota(jnp.int32, (BQ, BK), 1)
    s = jnp.where(col < K_REAL, s, -2.0)
    s_ref[...] = s
    m_ref[...] = jnp.max(s.reshape(BQ, BK // GSZ, GSZ), axis=2)


@jax.jit
def _k1(qn, tabn_pad):
    return pl.pallas_call(
        _k1_body,
        grid=(Q // BQ, NKC),
        in_specs=[
            pl.BlockSpec((BQ, DT), lambda i, j: (i, 0)),
            pl.BlockSpec((BK, DT), lambda i, j: (j, 0)),
        ],
        out_specs=[
            pl.BlockSpec((BQ, BK), lambda i, j: (i, j)),
            pl.BlockSpec((BQ, BK // GSZ), lambda i, j: (i, j)),
        ],
        out_shape=[
            jax.ShapeDtypeStruct((Q, K_PAD), jnp.float32),
            jax.ShapeDtypeStruct((Q, NG), jnp.float32),
        ],
        compiler_params=pltpu.CompilerParams(
            dimension_semantics=("arbitrary", "arbitrary"),
        ),
    )(qn, tabn_pad)


BQ2 = 64


def _k2_body(m_ref, m128_ref):
    mb = lax.bitcast_convert_type(m_ref[...], jnp.int32)
    mkey = jnp.where(mb >= 0, mb,
                     jnp.bitwise_xor(jnp.bitwise_not(mb), jnp.int32(-2**31)))

    def step(it, u):
        bit = 31 - it
        ucand = jnp.bitwise_or(u, jnp.int32(1) << bit)
        tcand = jnp.bitwise_xor(ucand, jnp.int32(-2**31))
        cnt = jnp.sum((mkey >= tcand).astype(jnp.int32), axis=1, keepdims=True)
        return jnp.where(cnt >= TOPK, ucand, u)

    u = lax.fori_loop(0, 32, step, jnp.zeros((BQ2, 1), jnp.int32))
    res = jnp.bitwise_xor(u, jnp.int32(-2**31))
    fb = jnp.where(res >= 0, res,
                   jnp.bitwise_not(jnp.bitwise_xor(res, jnp.int32(-2**31))))
    tau = lax.bitcast_convert_type(fb, jnp.float32)
    m128_ref[...] = jnp.concatenate(
        [m_ref[...], jnp.full((BQ2, MROW - NG - 1), -2.0, jnp.float32),
         tau], axis=1)


@jax.jit
def _k2(m):
    return pl.pallas_call(
        _k2_body,
        grid=(Q // BQ2,),
        in_specs=[pl.BlockSpec((BQ2, NG), lambda i: (i, 0))],
        out_specs=pl.BlockSpec((BQ2, MROW), lambda i: (i, 0)),
        out_shape=jax.ShapeDtypeStruct((Q, MROW), jnp.float32),
    )(m)


def _k3_make():
    mesh = plsc.VectorSubcoreMesh(core_axis_name="c", subcore_axis_name="s")

    @functools.partial(
        pl.kernel,
        mesh=mesh,
        compiler_params=pltpu.CompilerParams(needs_layout_passes=False),
        out_type=[
            jax.ShapeDtypeStruct((Q, CBUF), jnp.float32),
            jax.ShapeDtypeStruct((Q, CBUF), jnp.int32),
        ],
        scratch_types=[
            pltpu.VMEM((MROW,), jnp.float32),
            pltpu.VMEM((GL + 16,), jnp.int32),
            pltpu.VMEM((GL + 16,), jnp.int32),
            pltpu.VMEM((GL + 16, SG), jnp.float32),
            pltpu.VMEM((CBUF + 32,), jnp.float32),
            pltpu.VMEM((CBUF + 32,), jnp.int32),
            pltpu.SemaphoreType.DMA,
        ],
    )
    def k3(m_hbm, s2_hbm, cval_hbm, cidx_hbm,
           m_v, glist_v, garr_v, grow_v, cval_v, cidx_v, sem):
        wid = lax.axis_index("s") * 2 + lax.axis_index("c")
        q0 = wid * QPW
        iota = lax.iota(jnp.int32, 16)

        def per_query(i, _):
            q = q0 + i
            pltpu.sync_copy(m_hbm.at[q], m_v)
            tau_b = jnp.broadcast_to(m_v[pl.ds(MROW - 16, 16)][15], (16,))
            pad_row = 3136

            def prefill_g(j, _c):
                glist_v[pl.ds(j * 16, 16)] = jnp.full((16,), pad_row, jnp.int32)
                return _c
            lax.fori_loop(0, (GL + 16) // 16, prefill_g, 0)

            def scan_m(j, cnt):
                v = m_v[pl.ds(j * 16, 16)]
                msk = v >= tau_b
                gid = j * 16 + iota
                off = jnp.minimum(cnt, GL - 16)
                incl = plsc.cumsum(msk.astype(jnp.int32))
                pos = jnp.where(msk, off + incl - 1, GL + 8)
                plsc.store_scatter(glist_v, [pos], gid)
                npop = plsc.all_reduce_population_count(msk)[0]
                return cnt + npop
            lax.fori_loop(0, NG // 16, scan_m, jnp.int32(0))

            def mk_garr(jj, _c):
                gv = glist_v[pl.ds(jj * 16, 16)]
                garr_v[pl.ds(jj * 16, 16)] = (
                    q * NG2 + lax.shift_right_arithmetic(gv, 2))
                return _c
            lax.fori_loop(0, (GL + 16) // 16, mk_garr, 0)

            pltpu.async_copy(s2_hbm.at[garr_v], grow_v, sem).wait()

            def prefill_c(j, _c):
                cval_v[pl.ds(j * 16, 16)] = jnp.full((16,), -2.0, jnp.float32)
                cidx_v[pl.ds(j * 16, 16)] = jnp.zeros((16,), jnp.int32)
                return _c
            lax.fori_loop(0, (CBUF + 32) // 16, prefill_c, 0)

            def scan_rows(jj, ccnt):
                gvec = glist_v[pl.ds(jj * 16, 16)]
                cc = ccnt
                for l in range(16):
                    g32 = gvec[l]
                    qoff = (g32 & 3) * GSZ
                    row = jj * 16 + l
                    for h in range(GSZ // 16):
                        v = grow_v[row, pl.ds(qoff + h * 16, 16)]
                        msk = v >= tau_b
                        idxv = g32 * GSZ + h * 16 + iota
                        off = jnp.minimum(cc, CBUF)
                        incl = plsc.cumsum(msk.astype(jnp.int32))
                        pos = jnp.where(msk, off + incl - 1, CBUF + 24)
                        plsc.store_scatter(cval_v, [pos], v)
                        plsc.store_scatter(cidx_v, [pos], idxv)
                        cc = cc + plsc.all_reduce_population_count(msk)[0]
                return cc
            lax.fori_loop(0, GL // 16, scan_rows, jnp.int32(0))

            pltpu.sync_copy(cval_v.at[pl.ds(0, CBUF)], cval_hbm.at[q])
            pltpu.sync_copy(cidx_v.at[pl.ds(0, CBUF)], cidx_hbm.at[q])
            return _

        lax.fori_loop(0, QPW, per_query, 0)

    return k3


_K3 = None


def _get_k3():
    global _K3
    if _K3 is None:
        _K3 = _k3_make()
    return _K3


def _k4_body(cv_ref, ci_ref, ov_ref, oi_ref):
    v0 = cv_ref[...]
    ci = ci_ref[...]
    il = lax.broadcasted_iota(jnp.int32, (BQ, CBUF), 1)
    ol = lax.broadcasted_iota(jnp.int32, (BQ, 128), 1)

    def step(t, carry):
        v, ov, oi = carry
        m = jnp.max(v, axis=1, keepdims=True)
        p = jnp.min(jnp.where(v == m, il, CBUF), axis=1, keepdims=True)
        hit = il == p
        idx = jnp.sum(jnp.where(hit, ci, 0), axis=1, keepdims=True)
        v = jnp.where(hit, -3.0, v)
        ov = jnp.where(ol == t, m, ov)
        oi = jnp.where(ol == t, idx, oi)
        return v, ov, oi

    _, ov, oi = lax.fori_loop(
        0, TOPK, step,
        (v0, jnp.full((BQ, 128), -3.0, jnp.float32),
         jnp.zeros((BQ, 128), jnp.int32)))
    ov_ref[...] = ov
    oi_ref[...] = oi


@jax.jit
def _k4(cv, ci):
    return pl.pallas_call(
        _k4_body,
        grid=(Q // BQ,),
        in_specs=[
            pl.BlockSpec((BQ, CBUF), lambda i: (i, 0)),
            pl.BlockSpec((BQ, CBUF), lambda i: (i, 0)),
        ],
        out_specs=[
            pl.BlockSpec((BQ, 128), lambda i: (i, 0)),
            pl.BlockSpec((BQ, 128), lambda i: (i, 0)),
        ],
        out_shape=[
            jax.ShapeDtypeStruct((Q, 128), jnp.float32),
            jax.ShapeDtypeStruct((Q, 128), jnp.int32),
        ],
    )(cv, ci)


def kernel(query_embedding, movie_tag_embeddings, W, b, k):
    tabn = _l2n(movie_tag_embeddings)
    tabn_pad = jnp.concatenate(
        [tabn, jnp.zeros((K_PAD - K_REAL, DT), tabn.dtype)], axis=0)
    qn = _l2n(query_embedding @ W.T + b)
    scores, m = _k1(qn, tabn_pad)
    m128 = _k2(m)
    s2 = scores.reshape(Q * NG2, SG)
    cv, ci = _get_k3()(m128, s2)
    ov, oi = _k4(cv, ci)
    return ov[:, :TOPK], oi[:, :TOPK]


# speedup vs baseline: 14.3907x; 1.0026x over previous
import functools

import jax
import jax.numpy as jnp
from jax import lax
from jax.experimental import pallas as pl
from jax.experimental.pallas import tpu as pltpu
from jax.experimental.pallas import tpu_sc as plsc

Q = 4096
DQ = 384
DT = 128
K_REAL = 100000
GSZ = 32
NG = 3200
K_PAD = NG * GSZ
BQ = 256
BK = 4096
NKC = K_PAD // BK
TOPK = 100
CBUF = 512
GL = 128
SG = 128
NG2 = K_PAD // SG
NG2P = 896
MROW = 3328
NW = 32
QPW = Q // NW


def _l2n(x):
    n = jnp.sqrt(jnp.sum(x * x, axis=-1, keepdims=True))
    return x / jnp.maximum(n, 1e-12)


def _k1_body(qn_ref, tab_ref, s_ref, m_ref):
    j = pl.program_id(1)
    s = lax.dot_general(qn_ref[...], tab_ref[...], (((1,), (1,)), ((), ())),
                        preferred_element_type=jnp.float32)
    col = j * BK + lax.broadcasted_iota(jnp.int32, (BQ, BK), 1)
    s = jnp.where(col < K_REAL, s, -2.0)
    s_ref[...] = s
    m_ref[...] = jnp.max(s.reshape(BQ, BK // GSZ, GSZ), axis=2)


@jax.jit
def _k1(qn, tabn_pad):
    return pl.pallas_call(
        _k1_body,
        grid=(Q // BQ, NKC),
        in_specs=[
            pl.BlockSpec((BQ, DT), lambda i, j: (i, 0)),
            pl.BlockSpec((BK, DT), lambda i, j: (j, 0)),
        ],
        out_specs=[
            pl.BlockSpec((BQ, BK), lambda i, j: (i, j)),
            pl.BlockSpec((BQ, BK // GSZ), lambda i, j: (i, j)),
        ],
        out_shape=[
            jax.ShapeDtypeStruct((Q, K_PAD), jnp.float32),
            jax.ShapeDtypeStruct((Q, NG), jnp.float32),
        ],
        compiler_params=pltpu.CompilerParams(
            dimension_semantics=("arbitrary", "arbitrary"),
        ),
    )(qn, tabn_pad)


BQ2 = 64


def _k2_body(m_ref, m128_ref):
    mb = lax.bitcast_convert_type(m_ref[...], jnp.int32)
    mkey = jnp.where(mb >= 0, mb,
                     jnp.bitwise_xor(jnp.bitwise_not(mb), jnp.int32(-2**31)))

    def step(it, u):
        bit = 31 - it
        ucand = jnp.bitwise_or(u, jnp.int32(1) << bit)
        tcand = jnp.bitwise_xor(ucand, jnp.int32(-2**31))
        cnt = jnp.sum((mkey >= tcand).astype(jnp.int32), axis=1, keepdims=True)
        return jnp.where(cnt >= TOPK, ucand, u)

    u = lax.fori_loop(0, 32, step, jnp.zeros((BQ2, 1), jnp.int32))
    res = jnp.bitwise_xor(u, jnp.int32(-2**31))
    fb = jnp.where(res >= 0, res,
                   jnp.bitwise_not(jnp.bitwise_xor(res, jnp.int32(-2**31))))
    tau = lax.bitcast_convert_type(fb, jnp.float32)
    m128_ref[...] = jnp.concatenate(
        [m_ref[...], jnp.full((BQ2, MROW - NG - 1), -2.0, jnp.float32),
         tau], axis=1)


@jax.jit
def _k2(m):
    return pl.pallas_call(
        _k2_body,
        grid=(Q // BQ2,),
        in_specs=[pl.BlockSpec((BQ2, NG), lambda i: (i, 0))],
        out_specs=pl.BlockSpec((BQ2, MROW), lambda i: (i, 0)),
        out_shape=jax.ShapeDtypeStruct((Q, MROW), jnp.float32),
    )(m)


def _k3_make():
    mesh = plsc.VectorSubcoreMesh(core_axis_name="c", subcore_axis_name="s")
    D = 4

    @functools.partial(
        pl.kernel,
        mesh=mesh,
        compiler_params=pltpu.CompilerParams(needs_layout_passes=False),
        out_type=[
            jax.ShapeDtypeStruct((Q, CBUF), jnp.float32),
            jax.ShapeDtypeStruct((Q, CBUF), jnp.int32),
        ],
        scratch_types=[
            pltpu.VMEM((MROW,), jnp.float32),
            pltpu.VMEM((D * 16,), jnp.float32),
            [pltpu.VMEM((GL + 16,), jnp.int32) for _ in range(D)],
            [pltpu.VMEM((GL + 16,), jnp.int32) for _ in range(D)],
            [pltpu.VMEM((GL + 16, SG), jnp.float32) for _ in range(D)],
            pltpu.VMEM((CBUF + 32,), jnp.float32),
            pltpu.VMEM((CBUF + 32,), jnp.int32),
            [pltpu.SemaphoreType.DMA for _ in range(D)],
        ],
    )
    def k3(m_hbm, s2_hbm, cval_hbm, cidx_hbm,
           m_v, taus_v, glists, garrs, grows, cval_v, cidx_v, sems):
        wid = lax.axis_index("s") * 2 + lax.axis_index("c")
        q0 = wid * QPW
        iota = lax.iota(jnp.int32, 16)

        def prepare(q, d):
            glist_v, garr_v, grow_v, sem = glists[d], garrs[d], grows[d], sems[d]
            pltpu.sync_copy(m_hbm.at[q], m_v)
            tau_b = jnp.broadcast_to(m_v[pl.ds(MROW - 16, 16)][15], (16,))
            taus_v[pl.ds(d * 16, 16)] = tau_b

            def prefill_g(j, _c):
                glist_v[pl.ds(j * 16, 16)] = jnp.full((16,), 3136, jnp.int32)
                return _c
            lax.fori_loop(0, (GL + 16) // 16, prefill_g, 0)

            def scan_m(j, cnt):
                v = m_v[pl.ds(j * 16, 16)]
                msk = v >= tau_b
                gid = j * 16 + iota
                off = jnp.minimum(cnt, GL - 16)
                incl = plsc.cumsum(msk.astype(jnp.int32))
                pos = jnp.where(msk, off + incl - 1, GL + 8)
                plsc.store_scatter(glist_v, [pos], gid)
                npop = plsc.all_reduce_population_count(msk)[0]
                return cnt + npop
            lax.fori_loop(0, NG // 16, scan_m, jnp.int32(0))

            def mk_garr(jj, _c):
                gv = glist_v[pl.ds(jj * 16, 16)]
                garr_v[pl.ds(jj * 16, 16)] = (
                    q * NG2 + lax.shift_right_arithmetic(gv, 2))
                return _c
            lax.fori_loop(0, (GL + 16) // 16, mk_garr, 0)

            pltpu.async_copy(s2_hbm.at[garr_v], grow_v, sem)

        def process(q, d):
            glist_v, garr_v, grow_v, sem = glists[d], garrs[d], grows[d], sems[d]
            tau_b = taus_v[pl.ds(d * 16, 16)]
            pltpu.make_async_copy(s2_hbm.at[garr_v], grow_v, sem).wait()

            def prefill_c(j, _c):
                cval_v[pl.ds(j * 16, 16)] = jnp.full((16,), -2.0, jnp.float32)
                cidx_v[pl.ds(j * 16, 16)] = jnp.zeros((16,), jnp.int32)
                return _c
            lax.fori_loop(0, (CBUF + 32) // 16, prefill_c, 0)

            def scan_rows(jj, ccnt):
                gvec = glist_v[pl.ds(jj * 16, 16)]
                cc = ccnt
                for l in range(16):
                    g32 = gvec[l]
                    qoff = (g32 & 3) * GSZ
                    row = jj * 16 + l
                    for h in range(GSZ // 16):
                        v = grow_v[row, pl.ds(qoff + h * 16, 16)]
                        msk = v >= tau_b
                        idxv = g32 * GSZ + h * 16 + iota
                        off = jnp.minimum(cc, CBUF)
                        incl = plsc.cumsum(msk.astype(jnp.int32))
                        pos = jnp.where(msk, off + incl - 1, CBUF + 24)
                        plsc.store_scatter(cval_v, [pos], v)
                        plsc.store_scatter(cidx_v, [pos], idxv)
                        cc = cc + plsc.all_reduce_population_count(msk)[0]
                return cc
            lax.fori_loop(0, GL // 16, scan_rows, jnp.int32(0))

            pltpu.sync_copy(cval_v.at[pl.ds(0, CBUF)], cval_hbm.at[q])
            pltpu.sync_copy(cidx_v.at[pl.ds(0, CBUF)], cidx_hbm.at[q])

        for d in range(D):
            prepare(q0 + d, d)

        def outer(t, _):
            for d in range(D):
                i = t * D + d
                process(q0 + i, d)

                @pl.when(i + D < QPW)
                def _fire():
                    prepare(q0 + i + D, d)
            return _

        lax.fori_loop(0, QPW // D, outer, 0)

    return k3


_K3 = None


def _get_k3():
    global _K3
    if _K3 is None:
        _K3 = _k3_make()
    return _K3


def _k4_body(cv_ref, ci_ref, ov_ref, oi_ref):
    v0 = cv_ref[...]
    ci = ci_ref[...]
    il = lax.broadcasted_iota(jnp.int32, (BQ, CBUF), 1)
    ol = lax.broadcasted_iota(jnp.int32, (BQ, 128), 1)

    def step(t, carry):
        v, ov, oi = carry
        m = jnp.max(v, axis=1, keepdims=True)
        p = jnp.min(jnp.where(v == m, il, CBUF), axis=1, keepdims=True)
        hit = il == p
        idx = jnp.sum(jnp.where(hit, ci, 0), axis=1, keepdims=True)
        v = jnp.where(hit, -3.0, v)
        ov = jnp.where(ol == t, m, ov)
        oi = jnp.where(ol == t, idx, oi)
        return v, ov, oi

    _, ov, oi = lax.fori_loop(
        0, TOPK, step,
        (v0, jnp.full((BQ, 128), -3.0, jnp.float32),
         jnp.zeros((BQ, 128), jnp.int32)))
    ov_ref[...] = ov
    oi_ref[...] = oi


@jax.jit
def _k4(cv, ci):
    return pl.pallas_call(
        _k4_body,
        grid=(Q // BQ,),
        in_specs=[
            pl.BlockSpec((BQ, CBUF), lambda i: (i, 0)),
            pl.BlockSpec((BQ, CBUF), lambda i: (i, 0)),
        ],
        out_specs=[
            pl.BlockSpec((BQ, 128), lambda i: (i, 0)),
            pl.BlockSpec((BQ, 128), lambda i: (i, 0)),
        ],
        out_shape=[
            jax.ShapeDtypeStruct((Q, 128), jnp.float32),
            jax.ShapeDtypeStruct((Q, 128), jnp.int32),
        ],
    )(cv, ci)


def kernel(query_embedding, movie_tag_embeddings, W, b, k):
    tabn = _l2n(movie_tag_embeddings)
    tabn_pad = jnp.concatenate(
        [tabn, jnp.zeros((K_PAD - K_REAL, DT), tabn.dtype)], axis=0)
    qn = _l2n(query_embedding @ W.T + b)
    scores, m = _k1(qn, tabn_pad)
    m128 = _k2(m)
    s2 = scores.reshape(Q * NG2, SG)
    cv, ci = _get_k3()(m128, s2)
    ov, oi = _k4(cv, ci)
    return ov[:, :TOPK], oi[:, :TOPK]
